# baseline (speedup 1.0000x reference)
import functools

import jax
import jax.numpy as jnp
from jax import lax
from jax.experimental import pallas as pl
from jax.experimental.pallas import tpu as pltpu
from jax.experimental.pallas import tpu_sc as plsc

N_NODES = 10000
N_EDGES = 160000
LEAK = 0.1
BN_EPS = 1e-5

_SC_MESH = None


def _sc_mesh():
    global _SC_MESH
    if _SC_MESH is None:
        _SC_MESH = plsc.VectorSubcoreMesh(core_axis_name="c",
                                          subcore_axis_name="s")
    return _SC_MESH


_SC_PARAMS = pltpu.CompilerParams(use_tc_tiling_on_sc=False)


def _pad_nnz(col, dst, w, lanes_total, batch):
    nnz = col.shape[0]
    per = -(-nnz // (lanes_total * batch)) * batch
    npad = per * lanes_total
    pad = npad - nnz
    col = jnp.concatenate([col.astype(jnp.int32),
                           jnp.zeros((pad,), jnp.int32)])
    dst = jnp.concatenate([dst.astype(jnp.int32),
                           jnp.zeros((pad,), jnp.int32)])
    w = jnp.concatenate([w, jnp.zeros((pad,), jnp.float32)])
    return col, dst, w, per


def sc_segsum_node(x, col, dst, w, R, ones_mode=False):
    B = 128
    col, dst, w, per = _pad_nnz(col, dst, w, 32, B)
    F = 16 if ones_mode else x.shape[1]
    NB = per // B
    NF = F // 16
    S0 = (R // 16) // 8 * 8
    REM = R - 16 * S0

    @functools.partial(
        pl.kernel, mesh=_sc_mesh(), compiler_params=_SC_PARAMS,
        out_type=jax.ShapeDtypeStruct((2 * R, NF, 16), jnp.float32),
        scratch_types=[
            pltpu.VMEM((B,), jnp.int32),
            pltpu.VMEM((B,), jnp.int32),
            pltpu.VMEM((B,), jnp.float32),
            pltpu.VMEM((B, NF, 16), jnp.float32),
            pltpu.VMEM((S0 + REM, NF, 16), jnp.float32),
            pltpu.VMEM((16,), jnp.float32),
            pltpu.VMEM_SHARED((R, NF, 16), jnp.float32),
            pltpu.SemaphoreType.DMA,
        ])
    def k(x_hbm, col_hbm, dst_hbm, w_hbm, out_hbm,
          col_v, dst_v, w_v, rows_v, z_v, one_v, acc_sh, sem):
        sc = lax.axis_index("c")
        t = lax.axis_index("s")
        z_v[...] = jnp.zeros_like(z_v)
        one_v[...] = jnp.full((16,), 1.0, jnp.float32)
        pltpu.sync_copy(z_v.at[pl.ds(0, S0)], acc_sh.at[pl.ds(t * S0, S0)])
        @pl.when(t == 15)
        def _():
            pltpu.sync_copy(z_v.at[pl.ds(0, REM)],
                            acc_sh.at[pl.ds(16 * S0, REM)])
        plsc.subcore_barrier()

        base = (sc * 16 + t) * per

        def batch_body(g, carry):
            off = base + g * B
            pltpu.sync_copy(dst_hbm.at[pl.ds(off, B)], dst_v)
            pltpu.sync_copy(w_hbm.at[pl.ds(off, B)], w_v)
            if not ones_mode:
                pltpu.sync_copy(col_hbm.at[pl.ds(off, B)], col_v)
                pltpu.async_copy(x_hbm.at[col_v], rows_v, sem).wait()

            def mul_body(bh, c2):
                ws = w_v[pl.ds(bh * 16, 16)]
                for j in range(16):
                    b = bh * 16 + j
                    for f in range(NF):
                        if ones_mode:
                            rows_v[b, f, :] = one_v[...] * ws[j]
                        else:
                            rows_v[b, f, :] = rows_v[b, f, :] * ws[j]
                return c2
            lax.fori_loop(0, B // 16, mul_body, 0)
            pltpu.sync_copy(rows_v, acc_sh.at[dst_v], add=True)
            return carry
        lax.fori_loop(0, NB, batch_body, 0)

        plsc.subcore_barrier()
        pltpu.sync_copy(acc_sh.at[pl.ds(t * S0, S0)],
                        out_hbm.at[pl.ds(sc * R + t * S0, S0)])
        @pl.when(t == 15)
        def _():
            pltpu.sync_copy(acc_sh.at[pl.ds(16 * S0, REM)],
                            out_hbm.at[pl.ds(sc * R + 16 * S0, REM)])

    if ones_mode:
        x = jnp.zeros((8, 1, 16), jnp.float32)
    else:
        x = x.reshape(x.shape[0], NF, 16)
    return k(x, col, dst, w).reshape(2 * R, F)


def sc_segsum_edge(x_fg, R_in, col, dst, w, R, C):
    B = 128
    col, dst, w, per = _pad_nnz(col, dst, w, 16, B)
    FG = x_fg.shape[0] // R_in
    RC = R // C
    NB = per // B
    S0 = (RC // 16) // 8 * 8
    REM = RC - 16 * S0
    CH = C // 2

    @functools.partial(
        pl.kernel, mesh=_sc_mesh(), compiler_params=_SC_PARAMS,
        out_type=jax.ShapeDtypeStruct((R, 16 * FG), jnp.float32),
        scratch_types=[
            pltpu.VMEM((B,), jnp.int32),
            pltpu.VMEM((B,), jnp.int32),
            pltpu.VMEM((B,), jnp.float32),
            pltpu.VMEM((B,), jnp.int32),
            pltpu.VMEM((B,), jnp.int32),
            pltpu.VMEM((B, 16), jnp.float32),
            pltpu.VMEM((S0 + REM, 16), jnp.float32),
            pltpu.VMEM_SHARED((RC, 16), jnp.float32),
            pltpu.SemaphoreType.DMA,
        ])
    def k(x_hbm, col_hbm, dst_hbm, w_hbm, out_hbm,
          col_v, dst_v, w_v, gidx_v, sidx_v, rows_v, z_v, acc_sh, sem):
        sc = lax.axis_index("c")
        t = lax.axis_index("s")
        z_v[...] = jnp.zeros_like(z_v)
        base = t * per
        for ci in range(CH):
            cc = sc * CH + ci
            lo = cc * RC
            for fg in range(FG):
                pltpu.sync_copy(z_v.at[pl.ds(0, S0)],
                                acc_sh.at[pl.ds(t * S0, S0)])
                @pl.when(t == 15)
                def _():
                    pltpu.sync_copy(z_v.at[pl.ds(0, REM)],
                                    acc_sh.at[pl.ds(16 * S0, REM)])
                plsc.subcore_barrier()

                def batch_body(g, carry):
                    off = base + g * B
                    pltpu.sync_copy(col_hbm.at[pl.ds(off, B)], col_v)
                    pltpu.sync_copy(dst_hbm.at[pl.ds(off, B)], dst_v)
                    pltpu.sync_copy(w_hbm.at[pl.ds(off, B)], w_v)

                    def prep_body(bh, c2):
                        sl = pl.ds(bh * 16, 16)
                        if FG > 1:
                            gidx_v[sl] = col_v[sl] + fg * R_in
                        d16 = dst_v[sl]
                        inr = (d16 >= lo) & (d16 < lo + RC)
                        sidx_v[sl] = jnp.where(inr, d16 - lo, 0)
                        w_m = jnp.where(inr, w_v[sl], 0.0)
                        w_v[sl] = w_m
                        return c2
                    lax.fori_loop(0, B // 16, prep_body, 0)
                    gref = gidx_v if FG > 1 else col_v
                    pltpu.async_copy(x_hbm.at[gref], rows_v, sem).wait()

                    def mul_body(bh, c2):
                        ws = w_v[pl.ds(bh * 16, 16)]
                        for j in range(16):
                            b = bh * 16 + j
                            rows_v[b, :] = rows_v[b, :] * ws[j]
                        return c2
                    lax.fori_loop(0, B // 16, mul_body, 0)
                    pltpu.sync_copy(rows_v, acc_sh.at[sidx_v], add=True)
                    return carry
                lax.fori_loop(0, NB, batch_body, 0)

                plsc.subcore_barrier()
                pltpu.sync_copy(
                    acc_sh.at[pl.ds(t * S0, S0)],
                    out_hbm.at[pl.ds(lo + t * S0, S0), pl.ds(fg * 16, 16)])
                @pl.when(t == 15)
                def _():
                    pltpu.sync_copy(
                        acc_sh.at[pl.ds(16 * S0, REM)],
                        out_hbm.at[pl.ds(lo + 16 * S0, REM),
                                   pl.ds(fg * 16, 16)])
                plsc.subcore_barrier()

    return k(x_fg, col, dst, w)


def sc_gather2(m, src, dst):
    B = 128
    E = src.shape[0]
    F = m.shape[1]
    per = -(-E // (32 * B)) * B
    Ep = per * 32
    if Ep != E:
        z = jnp.zeros((Ep - E,), src.dtype)
        src = jnp.concatenate([src, z])
        dst = jnp.concatenate([dst, z])
    NB = per // B

    @functools.partial(
        pl.kernel, mesh=_sc_mesh(), compiler_params=_SC_PARAMS,
        out_type=jax.ShapeDtypeStruct((Ep, F // 16, 16), jnp.float32),
        scratch_types=[
            pltpu.VMEM((B,), jnp.int32),
            pltpu.VMEM((B,), jnp.int32),
            pltpu.VMEM((B, F // 16, 16), jnp.float32),
            pltpu.VMEM((B, F // 16, 16), jnp.float32),
            pltpu.SemaphoreType.DMA,
            pltpu.SemaphoreType.DMA,
        ])
    def k(m_hbm, src_hbm, dst_hbm, out_hbm,
          si_v, di_v, ra_v, rb_v, sem_a, sem_b):
        sc = lax.axis_index("c")
        t = lax.axis_index("s")
        base = (sc * 16 + t) * per

        def body(g, carry):
            off = base + g * B
            pltpu.sync_copy(src_hbm.at[pl.ds(off, B)], si_v)
            pltpu.sync_copy(dst_hbm.at[pl.ds(off, B)], di_v)
            cpa = pltpu.async_copy(m_hbm.at[si_v], ra_v, sem_a)
            cpb = pltpu.async_copy(m_hbm.at[di_v], rb_v, sem_b)
            cpa.wait()
            cpb.wait()

            def add_body(b, c2):
                for f in range(F // 16):
                    ra_v[b, f, :] = ra_v[b, f, :] + rb_v[b, f, :]
                return c2
            lax.fori_loop(0, B, add_body, 0)
            pltpu.sync_copy(ra_v, out_hbm.at[pl.ds(off, B)])
            return carry
        lax.fori_loop(0, NB, body, 0)

    m3 = m.reshape(m.shape[0], F // 16, 16)
    return k(m3, src.astype(jnp.int32),
             dst.astype(jnp.int32)).reshape(Ep, F)[:E]


def _leaky(x):
    return jnp.where(x > 0, x, LEAK * x)


def _row_block(r):
    for cand in (4000, 2000, 1000, 500, 250, 125, 100, 50, 25, 10, 8):
        if r % cand == 0:
            return cand
    return r


def _as_tuple(v):
    if v is None:
        return ()
    return tuple(v) if isinstance(v, (tuple, list)) else (v,)


def _fused_body(nx, nsubs, has_b, nadd, has_adddiv, has_outdiv, act,
                want_stats, want_fg, *refs):
    i = 0
    xs = refs[i:i + nx]; i += nx
    srefs = []
    for k in range(nx):
        srefs.append(refs[i:i + nsubs[k]]); i += nsubs[k]
    ws = refs[i:i + nx]; i += nx
    b_ref = refs[i] if has_b else None; i += has_b
    add_refs = refs[i:i + nadd]; i += nadd
    adddiv_ref = refs[i] if has_adddiv else None; i += has_adddiv
    outdiv_ref = refs[i] if has_outdiv else None; i += has_outdiv
    y_ref = refs[i]; i += 1
    st_ref = refs[i] if want_stats else None; i += want_stats
    fg_ref = refs[i] if want_fg else None

    acc = None
    for k in range(nx):
        v = xs[k][...]
        if srefs[k]:
            s = srefs[k][0][...]
            for sr in srefs[k][1:]:
                s = s + sr[...]
            v = v - s
        t = jnp.dot(v, ws[k][...], preferred_element_type=jnp.float32)
        acc = t if acc is None else acc + t
    if has_b:
        acc = acc + b_ref[...]
    if nadd:
        a = add_refs[0][...]
        for ar in add_refs[1:]:
            a = a + ar[...]
        if has_adddiv:
            a = a / adddiv_ref[...]
        acc = acc + a
    if has_outdiv:
        acc = acc / outdiv_ref[...]
    if act == "relu":
        acc = jnp.maximum(acc, 0.0)
    elif act == "leaky":
        acc = _leaky(acc)
    y_ref[...] = acc
    if want_stats:
        s1 = jnp.sum(acc, axis=0)
        s2 = jnp.sum(acc * acc, axis=0)
        st_ref[...] = jnp.stack([s1, s2])[None]
    if want_fg:
        for f in range(fg_ref.shape[0]):
            fg_ref[f, :, :] = acc[:, f * 16:(f + 1) * 16]


def fused_rows(parts, b=None, add=None, add_rowdiv=None, out_rowdiv=None,
               act="none", want_stats=False, want_fg=False):
    R = parts[0][0].shape[0]
    N = parts[0][2].shape[1]
    BR = _row_block(R)
    G = R // BR
    in_specs = []
    args = []
    nsubs = []
    for (x, _s, _w) in parts:
        in_specs.append(pl.BlockSpec((BR, x.shape[1]), lambda g: (g, 0)))
        args.append(x)
    for (x, s, _w) in parts:
        subs = _as_tuple(s)
        nsubs.append(len(subs))
        for sv in subs:
            in_specs.append(pl.BlockSpec((BR, x.shape[1]), lambda g: (g, 0)))
            args.append(sv)
    for (_x, _s, w) in parts:
        in_specs.append(pl.BlockSpec(w.shape, lambda g: (0, 0)))
        args.append(w)
    if b is not None:
        b2 = b.reshape(1, N)
        in_specs.append(pl.BlockSpec((1, N), lambda g: (0, 0)))
        args.append(b2)
    adds = _as_tuple(add)
    for av in adds:
        in_specs.append(pl.BlockSpec((BR, N), lambda g: (g, 0)))
        args.append(av)
    if add_rowdiv is not None:
        in_specs.append(pl.BlockSpec((BR, 1), lambda g: (g, 0)))
        args.append(add_rowdiv.reshape(R, 1))
    if out_rowdiv is not None:
        in_specs.append(pl.BlockSpec((BR, 1), lambda g: (g, 0)))
        args.append(out_rowdiv.reshape(R, 1))
    out_shape = [jax.ShapeDtypeStruct((R, N), jnp.float32)]
    out_specs = [pl.BlockSpec((BR, N), lambda g: (g, 0))]
    if want_stats:
        out_shape.append(jax.ShapeDtypeStruct((G, 2, N), jnp.float32))
        out_specs.append(pl.BlockSpec((1, 2, N), lambda g: (g, 0, 0)))
    if want_fg:
        FG = N // 16
        out_shape.append(jax.ShapeDtypeStruct((FG, R, 16), jnp.float32))
        out_specs.append(pl.BlockSpec((FG, BR, 16), lambda g: (0, g, 0)))
    body = functools.partial(
        _fused_body, len(parts), tuple(nsubs),
        b is not None, len(adds), add_rowdiv is not None,
        out_rowdiv is not None, act, want_stats, want_fg)
    single = not (want_stats or want_fg)
    res = pl.pallas_call(
        body,
        grid=(G,),
        in_specs=in_specs,
        out_specs=out_specs[0] if single else out_specs,
        out_shape=out_shape[0] if single else out_shape,
    )(*args)
    return res


def _bnfin_body(nrows, g_ref, bb_ref, st_ref, out_ref):
    s = jnp.sum(st_ref[...], axis=0)
    mean = s[0] / nrows
    var = s[1] / nrows - mean * mean
    scale = g_ref[...][0] / jnp.sqrt(var + BN_EPS)
    shift = bb_ref[...][0] - mean * scale
    out_ref[...] = jnp.stack([scale, shift])


def bn_finalize(stats, g, bb, nrows):
    G, _, N = stats.shape
    return pl.pallas_call(
        functools.partial(_bnfin_body, float(nrows)),
        in_specs=[pl.BlockSpec((1, N), lambda: (0, 0)),
                  pl.BlockSpec((1, N), lambda: (0, 0)),
                  pl.BlockSpec((G, 2, N), lambda: (0, 0, 0))],
        out_specs=pl.BlockSpec((2, N), lambda: (0, 0)),
        out_shape=jax.ShapeDtypeStruct((2, N), jnp.float32),
    )(g.reshape(1, N), bb.reshape(1, N), stats)


def _bnapply_body(want_fg, y_ref, ss_ref, out_ref, *rest):
    ss = ss_ref[...]
    acc = _leaky(y_ref[...] * ss[0] + ss[1])
    out_ref[...] = acc
    if want_fg:
        fg_ref = rest[0]
        for f in range(fg_ref.shape[0]):
            fg_ref[f, :, :] = acc[:, f * 16:(f + 1) * 16]


def bn_apply_leaky(y, ss, want_fg=False):
    R, N = y.shape
    BR = _row_block(R)
    out_shape = [jax.ShapeDtypeStruct((R, N), jnp.float32)]
    out_specs = [pl.BlockSpec((BR, N), lambda g: (g, 0))]
    if want_fg:
        FG = N // 16
        out_shape.append(jax.ShapeDtypeStruct((FG, R, 16), jnp.float32))
        out_specs.append(pl.BlockSpec((FG, BR, 16), lambda g: (0, g, 0)))
    return pl.pallas_call(
        functools.partial(_bnapply_body, want_fg),
        grid=(R // BR,),
        in_specs=[pl.BlockSpec((BR, N), lambda g: (g, 0)),
                  pl.BlockSpec((2, N), lambda g: (0, 0))],
        out_specs=out_specs[0] if not want_fg else out_specs,
        out_shape=out_shape[0] if not want_fg else out_shape,
    )(y, ss)


def _degfin_body(n, p_ref, out_ref):
    out_ref[...] = (p_ref[0:n, 0:1] + p_ref[n:2 * n, 0:1]) + 1e-6


def deg_finalize(partials, n):
    return pl.pallas_call(
        functools.partial(_degfin_body, n),
        in_specs=[pl.BlockSpec(partials.shape, lambda: (0, 0))],
        out_specs=pl.BlockSpec((n, 1), lambda: (0, 0)),
        out_shape=jax.ShapeDtypeStruct((n, 1), jnp.float32),
    )(partials)


def _dot_body(r_ref, w_ref, out_ref):
    @pl.when(pl.program_id(0) == 0)
    def _init():
        out_ref[...] = jnp.zeros_like(out_ref)
    out_ref[...] += jnp.sum(r_ref[...] * w_ref[...]).reshape(1, 1)


def big_dot(r, w):
    R = r.shape[0]
    BR = _row_block(R)
    return pl.pallas_call(
        _dot_body,
        grid=(R // BR,),
        in_specs=[pl.BlockSpec((BR, 1), lambda g: (g, 0)),
                  pl.BlockSpec((BR, 1), lambda g: (g, 0))],
        out_specs=pl.BlockSpec((1, 1), lambda g: (0, 0)),
        out_shape=jax.ShapeDtypeStruct((1, 1), jnp.float32),
    )(r.reshape(R, 1), w.reshape(R, 1))


def _pad_cols(a, width):
    r, c = a.shape
    if c == width:
        return a
    return jnp.concatenate([a, jnp.zeros((r, width - c), a.dtype)], axis=1)


def _pad_rows(w, k):
    if w.shape[0] == k:
        return w
    return jnp.concatenate(
        [w, jnp.zeros((k - w.shape[0], w.shape[1]), w.dtype)], axis=0)


def conv_block_node(x, ei, ew, p):
    N, F = x.shape
    s2 = sc_segsum_node(x, ei[1], ei[0], ew, N)
    W0 = _pad_rows(p['W'][0], F)
    W1 = _pad_rows(p['W'][1], F)
    y, st = fused_rows([(x, None, W0), (x, (s2[:N], s2[N:]), W1)],
                       b=p['b'], want_stats=True)
    ss = bn_finalize(st, p['g'], p['bb'], N)
    return bn_apply_leaky(y, ss)


def conv_block_edge(x, x_fg, ei, ew, p, C=8):
    E, F = x.shape
    s = sc_segsum_edge(x_fg, E, ei[1], ei[0], ew, E, C)
    W0 = _pad_rows(p['W'][0], F)
    W1 = _pad_rows(p['W'][1], F)
    y, st = fused_rows([(x, None, W0), (x, s, W1)], b=p['b'], want_stats=True)
    ss = bn_finalize(st, p['g'], p['bb'], E)
    return bn_apply_leaky(y, ss)


def kernel(x_t, x_s, edge_index, edge_index_t, edge_weight_t,
           edge_index_s, edge_weight_s, params):
    src = edge_index[0].astype(jnp.int32)
    dst = edge_index[1].astype(jnp.int32)
    p = params
    embW = _pad_cols(p['emb']['W'], 80)
    embb = jnp.concatenate([p['emb']['b'], jnp.zeros((6,), jnp.float32)])
    xt = fused_rows([(x_t[:, 1:], None, embW)], b=embb, act="relu")
    xt = conv_block_node(xt, edge_index_t, edge_weight_t, p['init_t'])

    xs_in = _pad_cols(x_s[:, 1:], 16)
    xs = conv_block_edge(xs_in, xs_in, edge_index_s, edge_weight_s,
                         p['init_s'])
    xt0, xs0 = xt, xs

    e_iota = jnp.arange(N_EDGES, dtype=jnp.int32)
    both_idx = jnp.concatenate([src, dst])
    both_e = jnp.concatenate([e_iota, e_iota])
    ones2e = jnp.ones((2 * N_EDGES,), jnp.float32)
    degp = sc_segsum_node(None, both_e, both_idx, ones2e, N_NODES,
                          ones_mode=True)
    deg = deg_finalize(degp, N_NODES)

    for i in range(3):
        q = p['neint%d' % i]
        m_s = fused_rows([(xs0, None, q['Wst'])])
        nfep = sc_segsum_node(m_s, both_e, both_idx, ones2e, N_NODES)
        xt_n = fused_rows([(xt0, None, q['Wtt'])],
                          add=(nfep[:N_NODES], nfep[N_NODES:]),
                          add_rowdiv=deg, act="leaky")
        m_t = fused_rows([(xt0, None, q['Wts'])], out_rowdiv=deg)
        xs_n, xs_n_fg = fused_rows([(xs0, None, q['Wss'])],
                                   add=sc_gather2(m_t, src, dst),
                                   act="leaky", want_fg=True)
        xt = conv_block_node(xt_n, edge_index_t, edge_weight_t,
                             p['nect%d' % i])
        xs = conv_block_edge(xs_n, xs_n_fg.reshape(-1, 16),
                             edge_index_s, edge_weight_s,
                             p['necs%d' % i])
        xt0 = jnp.concatenate([xt0, xt], -1)
        xs0 = jnp.concatenate([xs0, xs], -1)
    rt = fused_rows([(xt, None, p['ro_t']['W'][0])], b=p['ro_t']['b'])
    rs = fused_rows([(xs, None, p['ro_s']['W'][0])], b=p['ro_s']['b'])
    wv = p['out']['W'][:, 0]
    tot = (big_dot(rs, wv[:N_EDGES]) + big_dot(rt, wv[N_EDGES:])
           + p['out']['b'])
    return tot.reshape(1, 1)


# speedup vs baseline: 1.0992x; 1.0992x over previous
import functools

import jax
import jax.numpy as jnp
from jax import lax
from jax.experimental import pallas as pl
from jax.experimental.pallas import tpu as pltpu
from jax.experimental.pallas import tpu_sc as plsc

N_NODES = 10000
N_EDGES = 160000
LEAK = 0.1
BN_EPS = 1e-5

_SC_MESH = None


def _sc_mesh():
    global _SC_MESH
    if _SC_MESH is None:
        _SC_MESH = plsc.VectorSubcoreMesh(core_axis_name="c",
                                          subcore_axis_name="s")
    return _SC_MESH


_SC_PARAMS = pltpu.CompilerParams(use_tc_tiling_on_sc=False)


def _pad_nnz(col, dst, w, lanes_total, batch):
    nnz = col.shape[0]
    per = -(-nnz // (lanes_total * batch)) * batch
    npad = per * lanes_total
    pad = npad - nnz
    col = jnp.concatenate([col.astype(jnp.int32),
                           jnp.zeros((pad,), jnp.int32)])
    dst = jnp.concatenate([dst.astype(jnp.int32),
                           jnp.zeros((pad,), jnp.int32)])
    w = jnp.concatenate([w, jnp.zeros((pad,), jnp.float32)])
    return col, dst, w, per


def sc_segsum_node(x, col, dst, w, R, ones_mode=False):
    B = 512
    Q = B // 128
    col, dst, w, per = _pad_nnz(col, dst, w, 32, B)
    F = 16 if ones_mode else x.shape[1]
    NB = per // B
    NF = F // 16
    S0 = (R // 16) // 8 * 8
    REM = R - 16 * S0
    PR = per // 16

    @functools.partial(
        pl.kernel, mesh=_sc_mesh(), compiler_params=_SC_PARAMS,
        out_type=jax.ShapeDtypeStruct((2 * R, NF, 16), jnp.float32),
        scratch_types=[
            pltpu.VMEM((B,), jnp.int32),
            pltpu.VMEM((B,), jnp.int32),
            pltpu.VMEM((B,), jnp.float32),
            pltpu.VMEM((B, NF, 16), jnp.float32),
            pltpu.VMEM((S0 + REM, NF, 16), jnp.float32),
            pltpu.VMEM((16,), jnp.float32),
            pltpu.VMEM_SHARED((R, NF, 16), jnp.float32),
            pltpu.SemaphoreType.DMA,
        ])
    def k(x_hbm, col_hbm, dst_hbm, w_hbm, out_hbm,
          colb_v, dstb_v, wb_v, rows_v, z_v, one_v, acc_sh, sem):
        sc = lax.axis_index("c")
        t = lax.axis_index("s")
        z_v[...] = jnp.zeros_like(z_v)
        one_v[...] = jnp.full((16,), 1.0, jnp.float32)
        pltpu.sync_copy(z_v.at[pl.ds(0, S0)], acc_sh.at[pl.ds(t * S0, S0)])
        @pl.when(t == 15)
        def _():
            pltpu.sync_copy(z_v.at[pl.ds(0, REM)],
                            acc_sh.at[pl.ds(16 * S0, REM)])
        plsc.subcore_barrier()

        base = (sc * 16 + t) * per

        def batch_body(g, carry):
            off = base + g * B
            pltpu.sync_copy(dst_hbm.at[pl.ds(off, B)], dstb_v)
            pltpu.sync_copy(w_hbm.at[pl.ds(off, B)], wb_v)
            if not ones_mode:
                pltpu.sync_copy(col_hbm.at[pl.ds(off, B)], colb_v)
                hs = [pltpu.async_copy(
                          x_hbm.at[colb_v.at[pl.ds(q * 128, 128)]],
                          rows_v.at[pl.ds(q * 128, 128)], sem)
                      for q in range(Q)]
                for h in hs:
                    h.wait()

            def mul_body(kk, c2):
                ws = wb_v[pl.ds(kk * 16, 16)]
                for j in range(16):
                    b = kk * 16 + j
                    for f in range(NF):
                        if ones_mode:
                            rows_v[b, f, :] = one_v[...] * ws[j]
                        else:
                            rows_v[b, f, :] = rows_v[b, f, :] * ws[j]
                return c2
            lax.fori_loop(0, B // 16, mul_body, 0)
            for q in range(Q):
                pltpu.sync_copy(rows_v.at[pl.ds(q * 128, 128)],
                                acc_sh.at[dstb_v.at[pl.ds(q * 128, 128)]],
                                add=True)
            return carry
        lax.fori_loop(0, NB, batch_body, 0)

        plsc.subcore_barrier()
        pltpu.sync_copy(acc_sh.at[pl.ds(t * S0, S0)],
                        out_hbm.at[pl.ds(sc * R + t * S0, S0)])
        @pl.when(t == 15)
        def _():
            pltpu.sync_copy(acc_sh.at[pl.ds(16 * S0, REM)],
                            out_hbm.at[pl.ds(sc * R + 16 * S0, REM)])

    if ones_mode:
        x = jnp.zeros((8, 1, 16), jnp.float32)
    else:
        x = x.reshape(x.shape[0], NF, 16)
    return k(x, col, dst, w).reshape(2 * R, F)


def sc_segsum_edge(x_fg, R_in, col, dst, w, R, C):
    B = 512
    Q = B // 128
    col, dst, w, per = _pad_nnz(col, dst, w, 16, B)
    FG = x_fg.shape[0] // R_in
    RC = R // C
    NB = per // B
    S0 = (RC // 16) // 8 * 8
    REM = RC - 16 * S0
    CH = C // 2

    @functools.partial(
        pl.kernel, mesh=_sc_mesh(), compiler_params=_SC_PARAMS,
        out_type=jax.ShapeDtypeStruct((R, 16 * FG), jnp.float32),
        scratch_types=[
            pltpu.VMEM((B,), jnp.int32),
            pltpu.VMEM((B,), jnp.int32),
            pltpu.VMEM((B,), jnp.float32),
            pltpu.VMEM((B,), jnp.int32),
            pltpu.VMEM((B,), jnp.int32),
            pltpu.VMEM((B,), jnp.float32),
            pltpu.VMEM((B, 16), jnp.float32),
            pltpu.VMEM((S0 + REM, 16), jnp.float32),
            pltpu.VMEM_SHARED((RC, 16), jnp.float32),
            pltpu.SemaphoreType.DMA,
        ])
    def k(x_hbm, col_hbm, dst_hbm, w_hbm, out_hbm,
          colb_v, dstb_v, wb_v, gidxb_v, sidxb_v, wmb_v, rows_v, z_v,
          acc_sh, sem):
        sc = lax.axis_index("c")
        t = lax.axis_index("s")
        z_v[...] = jnp.zeros_like(z_v)
        base = t * per
        for ci in range(CH):
            cc = sc * CH + ci
            lo = cc * RC
            for fg in range(FG):
                pltpu.sync_copy(z_v.at[pl.ds(0, S0)],
                                acc_sh.at[pl.ds(t * S0, S0)])
                @pl.when(t == 15)
                def _():
                    pltpu.sync_copy(z_v.at[pl.ds(0, REM)],
                                    acc_sh.at[pl.ds(16 * S0, REM)])
                plsc.subcore_barrier()

                def batch_body(g, carry):
                    off = base + g * B
                    pltpu.sync_copy(col_hbm.at[pl.ds(off, B)], colb_v)
                    pltpu.sync_copy(dst_hbm.at[pl.ds(off, B)], dstb_v)
                    pltpu.sync_copy(w_hbm.at[pl.ds(off, B)], wb_v)

                    def prep_body(kk, c2):
                        sl = pl.ds(kk * 16, 16)
                        if FG > 1:
                            gidxb_v[sl] = colb_v[sl] + fg * R_in
                        d16 = dstb_v[sl]
                        inr = (d16 >= lo) & (d16 < lo + RC)
                        sidxb_v[sl] = jnp.where(inr, d16 - lo, 0)
                        wmb_v[sl] = jnp.where(inr, wb_v[sl], 0.0)
                        return c2
                    lax.fori_loop(0, B // 16, prep_body, 0)
                    gref = gidxb_v if FG > 1 else colb_v
                    hs = [pltpu.async_copy(
                              x_hbm.at[gref.at[pl.ds(q * 128, 128)]],
                              rows_v.at[pl.ds(q * 128, 128)], sem)
                          for q in range(Q)]
                    for h in hs:
                        h.wait()

                    def mul_body(kk, c2):
                        ws = wmb_v[pl.ds(kk * 16, 16)]
                        for j in range(16):
                            b = kk * 16 + j
                            rows_v[b, :] = rows_v[b, :] * ws[j]
                        return c2
                    lax.fori_loop(0, B // 16, mul_body, 0)
                    for q in range(Q):
                        pltpu.sync_copy(
                            rows_v.at[pl.ds(q * 128, 128)],
                            acc_sh.at[sidxb_v.at[pl.ds(q * 128, 128)]],
                            add=True)
                    return carry
                lax.fori_loop(0, NB, batch_body, 0)

                plsc.subcore_barrier()
                pltpu.sync_copy(
                    acc_sh.at[pl.ds(t * S0, S0)],
                    out_hbm.at[pl.ds(lo + t * S0, S0), pl.ds(fg * 16, 16)])
                @pl.when(t == 15)
                def _():
                    pltpu.sync_copy(
                        acc_sh.at[pl.ds(16 * S0, REM)],
                        out_hbm.at[pl.ds(lo + 16 * S0, REM),
                                   pl.ds(fg * 16, 16)])
                plsc.subcore_barrier()

    return k(x_fg, col, dst, w)


def sc_gather2(m, src, dst):
    B = 512
    E = src.shape[0]
    F = m.shape[1]
    per = -(-E // (32 * B)) * B
    Ep = per * 32
    if Ep != E:
        z = jnp.zeros((Ep - E,), src.dtype)
        src = jnp.concatenate([src, z])
        dst = jnp.concatenate([dst, z])
    NB = per // B

    @functools.partial(
        pl.kernel, mesh=_sc_mesh(), compiler_params=_SC_PARAMS,
        out_type=jax.ShapeDtypeStruct((Ep, F // 16, 16), jnp.float32),
        scratch_types=[
            pltpu.VMEM((B,), jnp.int32),
            pltpu.VMEM((B,), jnp.int32),
            pltpu.VMEM((B, F // 16, 16), jnp.float32),
            pltpu.VMEM((B, F // 16, 16), jnp.float32),
            pltpu.SemaphoreType.DMA,
            pltpu.SemaphoreType.DMA,
        ])
    def k(m_hbm, src_hbm, dst_hbm, out_hbm,
          si_v, di_v, ra_v, rb_v, sem_a, sem_b):
        sc = lax.axis_index("c")
        t = lax.axis_index("s")
        base = (sc * 16 + t) * per

        def body(g, carry):
            off = base + g * B
            pltpu.sync_copy(src_hbm.at[pl.ds(off, B)], si_v)
            pltpu.sync_copy(dst_hbm.at[pl.ds(off, B)], di_v)
            hs = []
            for q in range(B // 128):
                hs.append(pltpu.async_copy(
                    m_hbm.at[si_v.at[pl.ds(q * 128, 128)]],
                    ra_v.at[pl.ds(q * 128, 128)], sem_a))
                hs.append(pltpu.async_copy(
                    m_hbm.at[di_v.at[pl.ds(q * 128, 128)]],
                    rb_v.at[pl.ds(q * 128, 128)], sem_b))
            for h in hs:
                h.wait()

            def add_body(b, c2):
                for f in range(F // 16):
                    ra_v[b, f, :] = ra_v[b, f, :] + rb_v[b, f, :]
                return c2
            lax.fori_loop(0, B, add_body, 0)
            pltpu.sync_copy(ra_v, out_hbm.at[pl.ds(off, B)])
            return carry
        lax.fori_loop(0, NB, body, 0)

    m3 = m.reshape(m.shape[0], F // 16, 16)
    return k(m3, src.astype(jnp.int32),
             dst.astype(jnp.int32)).reshape(Ep, F)[:E]


def _leaky(x):
    return jnp.where(x > 0, x, LEAK * x)


def _row_block(r):
    for cand in (4000, 2000, 1000, 500, 250, 125, 100, 50, 25, 10, 8):
        if r % cand == 0:
            return cand
    return r


def _as_tuple(v):
    if v is None:
        return ()
    return tuple(v) if isinstance(v, (tuple, list)) else (v,)


def _fused_body(nx, nsubs, has_b, nadd, has_adddiv, has_outdiv, act,
                want_stats, want_fg, *refs):
    i = 0
    xs = refs[i:i + nx]; i += nx
    srefs = []
    for k in range(nx):
        srefs.append(refs[i:i + nsubs[k]]); i += nsubs[k]
    ws = refs[i:i + nx]; i += nx
    b_ref = refs[i] if has_b else None; i += has_b
    add_refs = refs[i:i + nadd]; i += nadd
    adddiv_ref = refs[i] if has_adddiv else None; i += has_adddiv
    outdiv_ref = refs[i] if has_outdiv else None; i += has_outdiv
    y_ref = refs[i]; i += 1
    st_ref = refs[i] if want_stats else None; i += want_stats
    fg_ref = refs[i] if want_fg else None

    acc = None
    for k in range(nx):
        v = xs[k][...]
        if srefs[k]:
            s = srefs[k][0][...]
            for sr in srefs[k][1:]:
                s = s + sr[...]
            v = v - s
        t = jnp.dot(v, ws[k][...], preferred_element_type=jnp.float32)
        acc = t if acc is None else acc + t
    if has_b:
        acc = acc + b_ref[...]
    if nadd:
        a = add_refs[0][...]
        for ar in add_refs[1:]:
            a = a + ar[...]
        if has_adddiv:
            a = a / adddiv_ref[...]
        acc = acc + a
    if has_outdiv:
        acc = acc / outdiv_ref[...]
    if act == "relu":
        acc = jnp.maximum(acc, 0.0)
    elif act == "leaky":
        acc = _leaky(acc)
    y_ref[...] = acc
    if want_stats:
        s1 = jnp.sum(acc, axis=0)
        s2 = jnp.sum(acc * acc, axis=0)
        st_ref[...] = jnp.stack([s1, s2])[None]
    if want_fg:
        for f in range(fg_ref.shape[0]):
            fg_ref[f, :, :] = acc[:, f * 16:(f + 1) * 16]


def fused_rows(parts, b=None, add=None, add_rowdiv=None, out_rowdiv=None,
               act="none", want_stats=False, want_fg=False):
    R = parts[0][0].shape[0]
    N = parts[0][2].shape[1]
    BR = _row_block(R)
    G = R // BR
    in_specs = []
    args = []
    nsubs = []
    for (x, _s, _w) in parts:
        in_specs.append(pl.BlockSpec((BR, x.shape[1]), lambda g: (g, 0)))
        args.append(x)
    for (x, s, _w) in parts:
        subs = _as_tuple(s)
        nsubs.append(len(subs))
        for sv in subs:
            in_specs.append(pl.BlockSpec((BR, x.shape[1]), lambda g: (g, 0)))
            args.append(sv)
    for (_x, _s, w) in parts:
        in_specs.append(pl.BlockSpec(w.shape, lambda g: (0, 0)))
        args.append(w)
    if b is not None:
        b2 = b.reshape(1, N)
        in_specs.append(pl.BlockSpec((1, N), lambda g: (0, 0)))
        args.append(b2)
    adds = _as_tuple(add)
    for av in adds:
        in_specs.append(pl.BlockSpec((BR, N), lambda g: (g, 0)))
        args.append(av)
    if add_rowdiv is not None:
        in_specs.append(pl.BlockSpec((BR, 1), lambda g: (g, 0)))
        args.append(add_rowdiv.reshape(R, 1))
    if out_rowdiv is not None:
        in_specs.append(pl.BlockSpec((BR, 1), lambda g: (g, 0)))
        args.append(out_rowdiv.reshape(R, 1))
    out_shape = [jax.ShapeDtypeStruct((R, N), jnp.float32)]
    out_specs = [pl.BlockSpec((BR, N), lambda g: (g, 0))]
    if want_stats:
        out_shape.append(jax.ShapeDtypeStruct((G, 2, N), jnp.float32))
        out_specs.append(pl.BlockSpec((1, 2, N), lambda g: (g, 0, 0)))
    if want_fg:
        FG = N // 16
        out_shape.append(jax.ShapeDtypeStruct((FG, R, 16), jnp.float32))
        out_specs.append(pl.BlockSpec((FG, BR, 16), lambda g: (0, g, 0)))
    body = functools.partial(
        _fused_body, len(parts), tuple(nsubs),
        b is not None, len(adds), add_rowdiv is not None,
        out_rowdiv is not None, act, want_stats, want_fg)
    single = not (want_stats or want_fg)
    res = pl.pallas_call(
        body,
        grid=(G,),
        in_specs=in_specs,
        out_specs=out_specs[0] if single else out_specs,
        out_shape=out_shape[0] if single else out_shape,
    )(*args)
    return res


def _bnfin_body(nrows, g_ref, bb_ref, st_ref, out_ref):
    s = jnp.sum(st_ref[...], axis=0)
    mean = s[0] / nrows
    var = s[1] / nrows - mean * mean
    scale = g_ref[...][0] / jnp.sqrt(var + BN_EPS)
    shift = bb_ref[...][0] - mean * scale
    out_ref[...] = jnp.stack([scale, shift])


def bn_finalize(stats, g, bb, nrows):
    G, _, N = stats.shape
    return pl.pallas_call(
        functools.partial(_bnfin_body, float(nrows)),
        in_specs=[pl.BlockSpec((1, N), lambda: (0, 0)),
                  pl.BlockSpec((1, N), lambda: (0, 0)),
                  pl.BlockSpec((G, 2, N), lambda: (0, 0, 0))],
        out_specs=pl.BlockSpec((2, N), lambda: (0, 0)),
        out_shape=jax.ShapeDtypeStruct((2, N), jnp.float32),
    )(g.reshape(1, N), bb.reshape(1, N), stats)


def _bnapply_body(want_fg, y_ref, ss_ref, out_ref, *rest):
    ss = ss_ref[...]
    acc = _leaky(y_ref[...] * ss[0] + ss[1])
    out_ref[...] = acc
    if want_fg:
        fg_ref = rest[0]
        for f in range(fg_ref.shape[0]):
            fg_ref[f, :, :] = acc[:, f * 16:(f + 1) * 16]


def bn_apply_leaky(y, ss, want_fg=False):
    R, N = y.shape
    BR = _row_block(R)
    out_shape = [jax.ShapeDtypeStruct((R, N), jnp.float32)]
    out_specs = [pl.BlockSpec((BR, N), lambda g: (g, 0))]
    if want_fg:
        FG = N // 16
        out_shape.append(jax.ShapeDtypeStruct((FG, R, 16), jnp.float32))
        out_specs.append(pl.BlockSpec((FG, BR, 16), lambda g: (0, g, 0)))
    return pl.pallas_call(
        functools.partial(_bnapply_body, want_fg),
        grid=(R // BR,),
        in_specs=[pl.BlockSpec((BR, N), lambda g: (g, 0)),
                  pl.BlockSpec((2, N), lambda g: (0, 0))],
        out_specs=out_specs[0] if not want_fg else out_specs,
        out_shape=out_shape[0] if not want_fg else out_shape,
    )(y, ss)


def _degfin_body(n, p_ref, out_ref):
    out_ref[...] = (p_ref[0:n, 0:1] + p_ref[n:2 * n, 0:1]) + 1e-6


def deg_finalize(partials, n):
    return pl.pallas_call(
        functools.partial(_degfin_body, n),
        in_specs=[pl.BlockSpec(partials.shape, lambda: (0, 0))],
        out_specs=pl.BlockSpec((n, 1), lambda: (0, 0)),
        out_shape=jax.ShapeDtypeStruct((n, 1), jnp.float32),
    )(partials)


def _dot_body(r_ref, w_ref, out_ref):
    @pl.when(pl.program_id(0) == 0)
    def _init():
        out_ref[...] = jnp.zeros_like(out_ref)
    out_ref[...] += jnp.sum(r_ref[...] * w_ref[...]).reshape(1, 1)


def big_dot(r, w):
    R = r.shape[0]
    BR = _row_block(R)
    return pl.pallas_call(
        _dot_body,
        grid=(R // BR,),
        in_specs=[pl.BlockSpec((BR, 1), lambda g: (g, 0)),
                  pl.BlockSpec((BR, 1), lambda g: (g, 0))],
        out_specs=pl.BlockSpec((1, 1), lambda g: (0, 0)),
        out_shape=jax.ShapeDtypeStruct((1, 1), jnp.float32),
    )(r.reshape(R, 1), w.reshape(R, 1))


def _pad_cols(a, width):
    r, c = a.shape
    if c == width:
        return a
    return jnp.concatenate([a, jnp.zeros((r, width - c), a.dtype)], axis=1)


def _pad_rows(w, k):
    if w.shape[0] == k:
        return w
    return jnp.concatenate(
        [w, jnp.zeros((k - w.shape[0], w.shape[1]), w.dtype)], axis=0)


def conv_block_node(x, ei, ew, p):
    N, F = x.shape
    if F > 64:
        pieces = [sc_segsum_node(x[:, i:i + 64], ei[1], ei[0], ew, N)
                  for i in range(0, F, 64)]
        s2 = jnp.concatenate(pieces, axis=1)
    else:
        s2 = sc_segsum_node(x, ei[1], ei[0], ew, N)
    W0 = _pad_rows(p['W'][0], F)
    W1 = _pad_rows(p['W'][1], F)
    y, st = fused_rows([(x, None, W0), (x, (s2[:N], s2[N:]), W1)],
                       b=p['b'], want_stats=True)
    ss = bn_finalize(st, p['g'], p['bb'], N)
    return bn_apply_leaky(y, ss)


def conv_block_edge(x, x_fg, ei, ew, p, C=8):
    E, F = x.shape
    s = sc_segsum_edge(x_fg, E, ei[1], ei[0], ew, E, C)
    W0 = _pad_rows(p['W'][0], F)
    W1 = _pad_rows(p['W'][1], F)
    y, st = fused_rows([(x, None, W0), (x, s, W1)], b=p['b'], want_stats=True)
    ss = bn_finalize(st, p['g'], p['bb'], E)
    return bn_apply_leaky(y, ss)


def kernel(x_t, x_s, edge_index, edge_index_t, edge_weight_t,
           edge_index_s, edge_weight_s, params):
    src = edge_index[0].astype(jnp.int32)
    dst = edge_index[1].astype(jnp.int32)
    p = params
    embW = _pad_cols(p['emb']['W'], 80)
    embb = jnp.concatenate([p['emb']['b'], jnp.zeros((6,), jnp.float32)])
    xt = fused_rows([(x_t[:, 1:], None, embW)], b=embb, act="relu")
    xt = conv_block_node(xt, edge_index_t, edge_weight_t, p['init_t'])

    xs_in = _pad_cols(x_s[:, 1:], 16)
    xs = conv_block_edge(xs_in, xs_in, edge_index_s, edge_weight_s,
                         p['init_s'])
    xt0, xs0 = xt, xs

    e_iota = jnp.arange(N_EDGES, dtype=jnp.int32)
    both_idx = jnp.concatenate([src, dst])
    both_e = jnp.concatenate([e_iota, e_iota])
    ones2e = jnp.ones((2 * N_EDGES,), jnp.float32)
    degp = sc_segsum_node(None, both_e, both_idx, ones2e, N_NODES,
                          ones_mode=True)
    deg = deg_finalize(degp, N_NODES)

    for i in range(3):
        q = p['neint%d' % i]
        m_s = fused_rows([(xs0, None, q['Wst'])])
        nfep = sc_segsum_node(m_s, both_e, both_idx, ones2e, N_NODES)
        xt_n = fused_rows([(xt0, None, q['Wtt'])],
                          add=(nfep[:N_NODES], nfep[N_NODES:]),
                          add_rowdiv=deg, act="leaky")
        m_t = fused_rows([(xt0, None, q['Wts'])], out_rowdiv=deg)
        xs_n, xs_n_fg = fused_rows([(xs0, None, q['Wss'])],
                                   add=sc_gather2(m_t, src, dst),
                                   act="leaky", want_fg=True)
        xt = conv_block_node(xt_n, edge_index_t, edge_weight_t,
                             p['nect%d' % i])
        xs = conv_block_edge(xs_n, xs_n_fg.reshape(-1, 16),
                             edge_index_s, edge_weight_s,
                             p['necs%d' % i])
        xt0 = jnp.concatenate([xt0, xt], -1)
        xs0 = jnp.concatenate([xs0, xs], -1)
    rt = fused_rows([(xt, None, p['ro_t']['W'][0])], b=p['ro_t']['b'])
    rs = fused_rows([(xs, None, p['ro_s']['W'][0])], b=p['ro_s']['b'])
    wv = p['out']['W'][:, 0]
    tot = (big_dot(rs, wv[:N_EDGES]) + big_dot(rt, wv[N_EDGES:])
           + p['out']['b'])
    return tot.reshape(1, 1)
